# baseline (speedup 1.0000x reference)
import functools

import jax
import jax.numpy as jnp
import numpy as np
from jax import lax
from jax.experimental import pallas as pl
from jax.experimental.pallas import tpu as pltpu
from jax.experimental.pallas import tpu_sc as plsc

VOCAB = 1000000
EMBED = 64
MAX_LEN = 1024
B, L = 4096, 200
N = B * L

NC, NS = 2, 16
NW = NC * NS
LANES = 16

RD = 800
NU1 = VOCAB // RD
U1_PER_W = NU1 // NW
U1_MAIN = U1_PER_W * NW

CH = 128
NSUB = 4
NBUF = 4
LOOK = 2
NR2 = L // NBUF


def _positional_encoding():
    position = jnp.arange(MAX_LEN, dtype=jnp.float32)[:, None]
    div_term = jnp.exp(
        jnp.arange(0, EMBED, 2, dtype=jnp.float32) * (-(np.log(10000.0) / EMBED)))
    pe = jnp.zeros((MAX_LEN, EMBED), dtype=jnp.float32)
    pe = pe.at[:, 0::2].set(jnp.sin(position * div_term))
    pe = pe.at[:, 1::2].set(jnp.cos(position * div_term))
    return pe[:L]


_mesh = plsc.VectorSubcoreMesh(core_axis_name="c", subcore_axis_name="s")


@functools.partial(
    pl.kernel,
    out_type=jax.ShapeDtypeStruct((VOCAB, EMBED), jnp.float32),
    mesh=_mesh,
    scratch_types=[pltpu.SemaphoreType.DMA],
    compiler_params=pltpu.CompilerParams(use_tc_tiling_on_sc=True, needs_layout_passes=False),
)
def _depad_sc(tab_hbm, out_hbm, csem):
    wid = lax.axis_index("s") * NC + lax.axis_index("c")
    ubase = wid * U1_PER_W

    def fire(u):
        r0 = pl.multiple_of(u * RD, RD)
        pltpu.async_copy(
            tab_hbm.at[pl.ds(r0, RD)], out_hbm.at[pl.ds(r0, RD)], csem)

    def drain(_u):
        pltpu.make_async_copy(
            tab_hbm.at[pl.ds(0, RD)], out_hbm.at[pl.ds(0, RD)], csem).wait()

    def body(i, _):
        fire(ubase + i)
        return 0

    lax.fori_loop(0, U1_PER_W, body, 0, unroll=False)
    for t in range(NU1 - U1_MAIN):
        @pl.when(wid == t)
        def _():
            fire(U1_MAIN + t)

    def dbody(i, _):
        drain(i)
        return 0

    lax.fori_loop(0, U1_PER_W, dbody, 0, unroll=False)
    for t in range(NU1 - U1_MAIN):
        @pl.when(wid == t)
        def _():
            drain(0)


@functools.partial(
    pl.kernel,
    out_type=jax.ShapeDtypeStruct((B, L, NSUB, LANES), jnp.float32),
    mesh=_mesh,
    scratch_types=[
        pltpu.VMEM((NBUF, NSUB, 128), jnp.int32),
        pltpu.VMEM((NBUF, NSUB, CH, LANES), jnp.float32),
        pltpu.VMEM((NBUF, NSUB, LANES), jnp.float32),
    ] + [pltpu.SemaphoreType.DMA] * (2 * NBUF),
    compiler_params=pltpu.CompilerParams(use_tc_tiling_on_sc=False, needs_layout_passes=False),
)
def _lookup_sc(tab16_hbm, idx_hbm, pe_hbm, out_hbm, idx_v, rows_v, pe_v,
               *sems):
    gsem = sems[:NBUF]
    osem = sems[NBUF:]
    wid = lax.axis_index("s") * NC + lax.axis_index("c")

    def fire_gather(l, b):
        pltpu.sync_copy(idx_hbm.at[l, wid], idx_v.at[b])
        pltpu.sync_copy(pe_hbm.at[l], pe_v.at[b])
        for p in range(NSUB):
            pltpu.async_copy(
                tab16_hbm.at[idx_v.at[b, p]], rows_v.at[b, p], gsem[b])

    def wait_gather(b):
        for p in range(NSUB):
            pltpu.make_async_copy(
                tab16_hbm.at[idx_v.at[b, p]], rows_v.at[b, p], gsem[b]
            ).wait()

    def add_pe(b):
        pevs = [pe_v[b, p, :] for p in range(NSUB)]

        @plsc.parallel_loop(0, CH, 1, unroll=8)
        def body(k):
            for p in range(NSUB):
                plsc.addupdate(rows_v.at[b, p, k], pevs[p])

    def fire_out(l, b):
        for p in range(NSUB):
            pltpu.async_copy(
                rows_v.at[b, p],
                out_hbm.at[pl.ds(wid * CH, CH), l, p],
                osem[b],
            )

    def wait_out(b):
        for p in range(NSUB):
            pltpu.make_async_copy(
                rows_v.at[b, p], out_hbm.at[pl.ds(0, CH), 0, p], osem[b]
            ).wait()

    def step(l, b, wait_o, prefetch):
        wait_gather(b)
        add_pe(b)
        fire_out(l, b)
        if prefetch:
            bf = (b + LOOK) % NBUF
            if wait_o:
                wait_out(bf)
            fire_gather(l + LOOK, bf)

    for c0 in range(LOOK):
        fire_gather(c0, c0)
    for b in range(NBUF):
        step(b, b, wait_o=(b + LOOK >= NBUF), prefetch=True)

    def round_body(g, _):
        for b in range(NBUF):
            step(g * NBUF + b, b, wait_o=True, prefetch=True)
        return 0

    lax.fori_loop(1, NR2 - 1, round_body, 0, unroll=False)

    for b in range(NBUF):
        l = (NR2 - 1) * NBUF + b
        step(l, b, wait_o=True, prefetch=(b + LOOK < NBUF))
    for b in range(NBUF):
        wait_out(b)


def kernel(sequence, token_table):
    pe = _positional_encoding()
    pe4 = pe.reshape(L, NSUB, LANES)
    seqt = sequence.T.astype(jnp.int32).reshape(L, 32, CH)
    idx4 = (seqt[:, :, None, :] * 4
            + jnp.arange(4, dtype=jnp.int32)[None, None, :, None])
    tab_lin = _depad_sc(token_table)
    tab16 = tab_lin.reshape(VOCAB * 4, LANES)
    out4 = _lookup_sc(tab16, idx4, pe4)
    return out4.reshape(B, L, EMBED)


# speedup vs baseline: 14.2662x; 14.2662x over previous
import functools

import jax
import jax.numpy as jnp
import numpy as np
from jax import lax
from jax.experimental import pallas as pl
from jax.experimental.pallas import tpu as pltpu
from jax.experimental.pallas import tpu_sc as plsc

VOCAB = 1000000
EMBED = 64
MAX_LEN = 1024
B, L = 4096, 200
N = B * L

NC, NS = 2, 16
NW = NC * NS
ROWS_PER_W = N // NW
IDX_BLK = 128
CHUNK = 256
NSUB = CHUNK // IDX_BLK
NCH = ROWS_PER_W // CHUNK
NBUF = 4
LOOK = 2
NR = NCH // NBUF
LANES = 16
EJ = EMBED // LANES


def _positional_encoding():
    position = jnp.arange(MAX_LEN, dtype=jnp.float32)[:, None]
    div_term = jnp.exp(
        jnp.arange(0, EMBED, 2, dtype=jnp.float32) * (-(np.log(10000.0) / EMBED)))
    pe = jnp.zeros((MAX_LEN, EMBED), dtype=jnp.float32)
    pe = pe.at[:, 0::2].set(jnp.sin(position * div_term))
    pe = pe.at[:, 1::2].set(jnp.cos(position * div_term))
    return pe[:L]


_mesh = plsc.VectorSubcoreMesh(core_axis_name="c", subcore_axis_name="s")


@functools.partial(
    pl.kernel,
    out_type=jax.ShapeDtypeStruct((N, EMBED), jnp.float32),
    mesh=_mesh,
    scratch_types=[
        pltpu.VMEM((NBUF, NSUB, IDX_BLK), jnp.int32),
        pltpu.VMEM((NBUF, CHUNK, EMBED), jnp.float32),
        pltpu.VMEM((L, EMBED), jnp.float32),
    ] + [pltpu.SemaphoreType.DMA] * (2 * NBUF),
    compiler_params=pltpu.CompilerParams(use_tc_tiling_on_sc=False),
)
def _embed_sc(table_hbm, idx_hbm, pe_hbm, out_hbm, idx_v, rows_v, pe_v, *sems):
    gsem = sems[:NBUF]
    osem = sems[NBUF:]
    wid = lax.axis_index("s") * NC + lax.axis_index("c")
    base = wid * ROWS_PER_W

    pltpu.sync_copy(pe_hbm, pe_v)

    def fire_gather(c, b):
        pltpu.sync_copy(idx_hbm.at[wid * NCH + c], idx_v.at[b])
        for j in range(NSUB):
            pltpu.async_copy(
                table_hbm.at[idx_v.at[b, j]],
                rows_v.at[b, pl.ds(j * IDX_BLK, IDX_BLK)],
                gsem[b],
            )

    def wait_gather(b):
        for j in range(NSUB):
            pltpu.make_async_copy(
                table_hbm.at[idx_v.at[b, j]],
                rows_v.at[b, pl.ds(j * IDX_BLK, IDX_BLK)],
                gsem[b],
            ).wait()

    def add_pe(c, b):
        p0 = lax.rem(c * CHUNK, L)

        @plsc.parallel_loop(0, CHUNK, 1, unroll=4)
        def body(r):
            p = lax.rem(p0 + r, L)
            for j in range(EJ):
                v = pe_v[p, pl.ds(j * LANES, LANES)]
                plsc.addupdate(rows_v.at[b, r, pl.ds(j * LANES, LANES)], v)

    def fire_out(c, b):
        off = base + c * CHUNK
        pltpu.async_copy(rows_v.at[b], out_hbm.at[pl.ds(off, CHUNK)], osem[b])

    def wait_out(b):
        pltpu.make_async_copy(
            rows_v.at[b], out_hbm.at[pl.ds(base, CHUNK)], osem[b]
        ).wait()

    def step(c, b, wait_o, prefetch):
        wait_gather(b)
        add_pe(c, b)
        fire_out(c, b)
        if prefetch:
            bf = (b + LOOK) % NBUF
            if wait_o:
                wait_out(bf)
            fire_gather(c + LOOK, bf)

    for c0 in range(LOOK):
        fire_gather(c0, c0)

    for b in range(NBUF):
        step(b, b, wait_o=(b + LOOK >= NBUF), prefetch=True)

    def round_body(g, _):
        for b in range(NBUF):
            step(g * NBUF + b, b, wait_o=True, prefetch=True)
        return 0

    lax.fori_loop(1, NR - 1, round_body, 0, unroll=False)

    for b in range(NBUF):
        c = (NR - 1) * NBUF + b
        step(c, b, wait_o=True, prefetch=(c + LOOK < NCH))

    for b in range(NBUF):
        wait_out(b)


def kernel(sequence, token_table):
    pe = _positional_encoding()
    idx = sequence.reshape(N).astype(jnp.int32).reshape(N // CHUNK, NSUB, IDX_BLK)
    out = _embed_sc(token_table, idx, pe)
    return out.reshape(B, L, EMBED)
